# baseline (speedup 1.0000x reference)
import jax
import jax.numpy as jnp
from jax.experimental import pallas as pl


def _body(x_ref, w_ref, o_ref):
    w = w_ref[...]
    norms = jnp.sqrt(jnp.sum(w * w, axis=1, keepdims=True))
    emb = jnp.where(norms > 1.0, w * (1.0 / (norms + 1e-7)), w)
    o_ref[...] = x_ref[...] + emb[None, :, None, :]


def kernel(x, W):
    B, S, T, D = x.shape
    TB = 256
    grid = (B, T // TB)
    return pl.pallas_call(
        _body,
        grid=grid,
        in_specs=[
            pl.BlockSpec((1, S, TB, D), lambda i, j: (i, 0, j, 0)),
            pl.BlockSpec((S, D), lambda i, j: (0, 0)),
        ],
        out_specs=pl.BlockSpec((1, S, TB, D), lambda i, j: (i, 0, j, 0)),
        out_shape=jax.ShapeDtypeStruct(x.shape, x.dtype),
    )(x, W)


# speedup vs baseline: 1.0862x; 1.0862x over previous
import jax
import jax.numpy as jnp
from jax.experimental import pallas as pl


def _body(x_ref, w_ref, o_ref):
    w = w_ref[...]
    norms = jnp.sqrt(jnp.sum(w * w, axis=1, keepdims=True))
    emb = jnp.where(norms > 1.0, w * (1.0 / (norms + 1e-7)), w)
    o_ref[...] = x_ref[...] + emb[None, :, None, :]


def kernel(x, W):
    B, S, T, D = x.shape
    TB = 512
    grid = (B, T // TB)
    return pl.pallas_call(
        _body,
        grid=grid,
        in_specs=[
            pl.BlockSpec((1, S, TB, D), lambda i, j: (i, 0, j, 0)),
            pl.BlockSpec((S, D), lambda i, j: (0, 0)),
        ],
        out_specs=pl.BlockSpec((1, S, TB, D), lambda i, j: (i, 0, j, 0)),
        out_shape=jax.ShapeDtypeStruct(x.shape, x.dtype),
    )(x, W)


# speedup vs baseline: 1.1050x; 1.0173x over previous
import jax
import jax.numpy as jnp
from jax.experimental import pallas as pl


def _body(x_ref, w_ref, o_ref):
    w = w_ref[...]
    norms = jnp.sqrt(jnp.sum(w * w, axis=1, keepdims=True))
    emb = jnp.where(norms > 1.0, w * (1.0 / (norms + 1e-7)), w)
    o_ref[...] = x_ref[...] + emb[None, :, None, :]


def kernel(x, W):
    B, S, T, D = x.shape
    TB = 1024
    grid = (B, T // TB)
    return pl.pallas_call(
        _body,
        grid=grid,
        in_specs=[
            pl.BlockSpec((1, S, TB, D), lambda i, j: (i, 0, j, 0)),
            pl.BlockSpec((S, D), lambda i, j: (0, 0)),
        ],
        out_specs=pl.BlockSpec((1, S, TB, D), lambda i, j: (i, 0, j, 0)),
        out_shape=jax.ShapeDtypeStruct(x.shape, x.dtype),
    )(x, W)
